# baseline (speedup 1.0000x reference)
import functools

import jax
import jax.numpy as jnp
from jax import lax
from jax.experimental import pallas as pl
from jax.experimental.pallas import tpu as pltpu
from jax.experimental.pallas import tpu_sc as plsc

N = 100000
F = 128
B = 512
NV = F // 16
TILES = 16
CHUNK = 128
NCH = 49
PER_TILE = NCH * CHUNK
LAST_ROWS = N - 15 * PER_TILE
LAST_FULL = LAST_ROWS // CHUNK
LAST_REM = LAST_ROWS - LAST_FULL * CHUNK
ACC_ROWS = 528
ZROWS = B // TILES


def _prep_ids(batch):
    ids = batch.astype(jnp.int32)
    pad = jnp.full((TILES * PER_TILE - N,), B, jnp.int32)
    flat = jnp.concatenate([ids, pad])
    return flat.reshape(TILES, NCH, CHUNK), flat.reshape(TILES, PER_TILE)


def _sc_segment_sums(x_s, ids_s, fids_s, x_t, ids_t, fids_t,
                     zeros_f, ones_1, ident):
    mesh = plsc.VectorSubcoreMesh(core_axis_name="c", subcore_axis_name="s")

    @functools.partial(
        pl.kernel,
        out_type=(
            jax.ShapeDtypeStruct((B, F), jnp.float32),
            jax.ShapeDtypeStruct((B,), jnp.float32),
            jax.ShapeDtypeStruct((B, F), jnp.float32),
            jax.ShapeDtypeStruct((B,), jnp.float32),
        ),
        mesh=mesh,
        scratch_types=[
            pltpu.VMEM((NCH, CHUNK), jnp.int32),
            pltpu.VMEM((PER_TILE,), jnp.int32),
            pltpu.VMEM((2, CHUNK, F), jnp.float32),
            pltpu.VMEM((ACC_ROWS, F), jnp.float32),
            pltpu.VMEM((CHUNK,), jnp.float32),
            pltpu.VMEM((ZROWS,), jnp.float32),
            pltpu.VMEM((4, CHUNK), jnp.int32),
            pltpu.VMEM_SHARED((ACC_ROWS, F), jnp.float32),
            pltpu.VMEM_SHARED((ACC_ROWS,), jnp.float32),
            pltpu.SemaphoreType.DMA((2,)),
            pltpu.SemaphoreType.DMA,
        ],
    )
    def k(x_s_hbm, ids_s_hbm, fids_s_hbm, x_t_hbm, ids_t_hbm, fids_t_hbm,
          zf_hbm, ones_hbm, ident_hbm,
          sum_s_hbm, cnt_s_hbm, sum_t_hbm, cnt_t_hbm,
          idx_v, fids_v, data_v, acc_v, ones_v, c32_v, idv,
          acc_sh, cnt_sh, ld_sems, cnt_sem):
        c = lax.axis_index("c")
        s = lax.axis_index("s")
        pltpu.sync_copy(zf_hbm, acc_sh.at[pl.ds(s * ZROWS, ZROWS)])
        c32_v[pl.ds(0, 16)] = jnp.zeros((16,), jnp.float32)
        c32_v[pl.ds(16, 16)] = jnp.zeros((16,), jnp.float32)
        pltpu.sync_copy(c32_v, cnt_sh.at[pl.ds(s * ZROWS, ZROWS)])
        pltpu.sync_copy(ones_hbm, ones_v)
        pltpu.sync_copy(ident_hbm, idv)
        plsc.subcore_barrier()
        zvec = jnp.zeros((16,), jnp.float32)

        def side(x_hbm, ids_hbm, fids_hbm, sum_hbm, cnt_hbm):
            pltpu.sync_copy(ids_hbm.at[s], idx_v)
            pltpu.sync_copy(fids_hbm.at[s], fids_v)
            base = s * PER_TILE

            def zbody(i, carry):
                for j in range(NV):
                    acc_v[i, pl.ds(j * 16, 16)] = zvec
                return carry
            lax.fori_loop(0, ACC_ROWS, zbody, 0)

            def ld(ci, b):
                return pltpu.make_async_copy(
                    x_hbm.at[pl.ds(base + ci * CHUNK, CHUNK)],
                    data_v.at[b], ld_sems.at[b])

            def cnt_start(ci):
                pltpu.async_copy(ones_v, cnt_sh.at[idx_v.at[ci]], cnt_sem,
                                 add=True)

            def cnt_wait(ci):
                pltpu.make_async_copy(ones_v, cnt_sh.at[idx_v.at[ci]],
                                      cnt_sem).wait()

            def rows16(ci, b, g, carry):
                accs, prev = carry[:NV], carry[NV]
                accs = list(accs)
                iv = fids_v[pl.ds(ci * CHUNK + g * 16, 16)]
                for l in range(16):
                    sid = iv[l]
                    keep = jnp.full(
                        (16,), jnp.where(sid == prev, 1.0, 0.0))
                    r = g * 16 + l
                    for j in range(NV):
                        w = data_v[b, r, pl.ds(j * 16, 16)]
                        accs[j] = w + keep * accs[j]
                    for j in range(NV):
                        acc_v[sid, pl.ds(j * 16, 16)] = accs[j]
                    prev = sid
                return (*accs, prev)

            def chunk_body(nfull):
                def body(ci, carry):
                    b = ci & 1
                    ld(ci, b).wait()
                    cnt_start(ci)

                    def gbody(g, cr):
                        return rows16(ci, b, g, cr)

                    carry = lax.fori_loop(0, CHUNK // 16, gbody, carry)

                    @pl.when(ci + 2 < nfull)
                    def _():
                        ld(ci + 2, b).start()

                    return carry
                return body

            init = (*([zvec] * NV), jnp.int32(-1))

            ld(0, 0).start()
            ld(1, 1).start()

            @pl.when(s < TILES - 1)
            def _():
                lax.fori_loop(0, NCH, chunk_body(NCH), init)

            @pl.when(s == TILES - 1)
            def _():
                cr = lax.fori_loop(0, LAST_FULL, chunk_body(LAST_FULL), init)
                pltpu.sync_copy(
                    x_hbm.at[pl.ds(base + LAST_FULL * CHUNK, LAST_REM)],
                    data_v.at[0, pl.ds(0, LAST_REM)])

                def tbody(g, cr2):
                    return rows16(LAST_FULL, 0, g, cr2)

                lax.fori_loop(0, LAST_REM // 16, tbody, cr)
                cnt_start(LAST_FULL)

            def comb(gg, carry):
                pltpu.sync_copy(acc_v.at[pl.ds(gg * CHUNK, CHUNK)],
                                acc_sh.at[idv.at[gg]], add=True)
                return carry
            lax.fori_loop(0, B // CHUNK, comb, 0)

            def cdrain(nfull):
                def dbody(j, carry):
                    cnt_wait(j)
                    return carry
                lax.fori_loop(0, nfull, dbody, 0)

            @pl.when(s < TILES - 1)
            def _():
                cdrain(NCH)

            @pl.when(s == TILES - 1)
            def _():
                cdrain(LAST_FULL + 1)

            plsc.subcore_barrier()
            pltpu.sync_copy(acc_sh.at[pl.ds(s * ZROWS, ZROWS)],
                            sum_hbm.at[pl.ds(s * ZROWS, ZROWS)])
            pltpu.sync_copy(cnt_sh.at[pl.ds(s * ZROWS, ZROWS)], c32_v)
            pltpu.sync_copy(c32_v, cnt_hbm.at[pl.ds(s * ZROWS, ZROWS)])

        @pl.when(c == 0)
        def _():
            side(x_s_hbm, ids_s_hbm, fids_s_hbm, sum_s_hbm, cnt_s_hbm)

        @pl.when(c == 1)
        def _():
            side(x_t_hbm, ids_t_hbm, fids_t_hbm, sum_t_hbm, cnt_t_hbm)

    return k(x_s, ids_s, fids_s, x_t, ids_t, fids_t, zeros_f, ones_1, ident)


def _tc_mlp(sum_s, cnt_s, sum_t, cnt_t, u, w1t, b1, w2t, b2):
    def body(sum_s_ref, cnt_s_ref, sum_t_ref, cnt_t_ref, u_ref,
             w1_ref, b1_ref, w2_ref, b2_ref, out_ref):
        cs = jnp.maximum(cnt_s_ref[...], 1.0)
        ct = jnp.maximum(cnt_t_ref[...], 1.0)
        ms = sum_s_ref[...] / cs
        mt = sum_t_ref[...] / ct
        h = (jnp.dot(u_ref[...], w1_ref[0:F],
                     preferred_element_type=jnp.float32)
             + jnp.dot(ms, w1_ref[F:2 * F],
                       preferred_element_type=jnp.float32)
             + jnp.dot(mt, w1_ref[2 * F:3 * F],
                       preferred_element_type=jnp.float32)
             + b1_ref[...])
        h = jnp.where(h >= 0, h, 0.1 * h)
        out_ref[...] = (jnp.dot(h, w2_ref[...],
                                preferred_element_type=jnp.float32)
                        + b2_ref[...])

    return pl.pallas_call(
        body,
        out_shape=jax.ShapeDtypeStruct((B, F), jnp.float32),
    )(sum_s, cnt_s, sum_t, cnt_t, u, w1t, b1, w2t, b2)


def kernel(x_s, x_t, edge_index, edge_attr, u, batch_s, batch_t, W1, b1, W2, b2):
    del edge_index, edge_attr
    ids_s, fids_s = _prep_ids(batch_s)
    ids_t, fids_t = _prep_ids(batch_t)
    zeros_f = jnp.zeros((ZROWS, F), jnp.float32)
    ones_1 = jnp.ones((CHUNK,), jnp.float32)
    ident = jnp.arange(B, dtype=jnp.int32).reshape(B // CHUNK, CHUNK)
    sum_s, cnt_s, sum_t, cnt_t = _sc_segment_sums(
        x_s, ids_s, fids_s, x_t, ids_t, fids_t, zeros_f, ones_1, ident)
    return _tc_mlp(sum_s, cnt_s.reshape(B, 1), sum_t, cnt_t.reshape(B, 1), u,
                   W1.T, b1.reshape(1, F), W2.T, b2.reshape(1, F))


# speedup vs baseline: 1.0590x; 1.0590x over previous
import functools

import jax
import jax.numpy as jnp
from jax import lax
from jax.experimental import pallas as pl
from jax.experimental.pallas import tpu as pltpu
from jax.experimental.pallas import tpu_sc as plsc

N = 100000
F = 128
B = 512
TILES = 16
CHUNK = 128
NCH = 49
PER_TILE = NCH * CHUNK
LAST_ROWS = N - 15 * PER_TILE
LAST_FULL = LAST_ROWS // CHUNK
LAST_REM = LAST_ROWS - LAST_FULL * CHUNK
ACC_ROWS = 528
ZROWS = B // TILES


def _prep_ids(batch):
    ids = batch.astype(jnp.int32)
    pad = jnp.full((TILES * PER_TILE - N,), B, jnp.int32)
    return jnp.concatenate([ids, pad]).reshape(TILES, NCH, CHUNK)


def _sc_segment_sums(x_s, ids_s, x_t, ids_t, zeros_f, ones_1):
    mesh = plsc.VectorSubcoreMesh(core_axis_name="c", subcore_axis_name="s")

    @functools.partial(
        pl.kernel,
        out_type=(
            jax.ShapeDtypeStruct((B, F), jnp.float32),
            jax.ShapeDtypeStruct((B,), jnp.float32),
            jax.ShapeDtypeStruct((B, F), jnp.float32),
            jax.ShapeDtypeStruct((B,), jnp.float32),
        ),
        mesh=mesh,
        scratch_types=[
            pltpu.VMEM((NCH, CHUNK), jnp.int32),
            pltpu.VMEM((4, CHUNK, F), jnp.float32),
            pltpu.VMEM((CHUNK,), jnp.float32),
            pltpu.VMEM((ZROWS,), jnp.float32),
            pltpu.VMEM_SHARED((ACC_ROWS, F), jnp.float32),
            pltpu.VMEM_SHARED((ACC_ROWS,), jnp.float32),
            pltpu.SemaphoreType.DMA((4,)),
            pltpu.SemaphoreType.DMA((4,)),
            pltpu.SemaphoreType.DMA,
        ],
    )
    def k(x_s_hbm, ids_s_hbm, x_t_hbm, ids_t_hbm, zf_hbm, ones_hbm,
          sum_s_hbm, cnt_s_hbm, sum_t_hbm, cnt_t_hbm,
          idx_v, data_v, ones_v, c32_v, acc_sh, cnt_sh,
          ld_sems, sc_sems, cnt_sem):
        c = lax.axis_index("c")
        s = lax.axis_index("s")
        pltpu.sync_copy(zf_hbm, acc_sh.at[pl.ds(s * ZROWS, ZROWS)])
        c32_v[pl.ds(0, 16)] = jnp.zeros((16,), jnp.float32)
        c32_v[pl.ds(16, 16)] = jnp.zeros((16,), jnp.float32)
        pltpu.sync_copy(c32_v, cnt_sh.at[pl.ds(s * ZROWS, ZROWS)])
        pltpu.sync_copy(ones_hbm, ones_v)
        plsc.subcore_barrier()

        def side(x_hbm, ids_hbm, sum_hbm, cnt_hbm):
            pltpu.sync_copy(ids_hbm.at[s], idx_v)
            base = s * PER_TILE

            def ld(ci, b):
                return pltpu.make_async_copy(
                    x_hbm.at[pl.ds(base + ci * CHUNK, CHUNK)],
                    data_v.at[b], ld_sems.at[b])

            class _Cp:
                def __init__(self, src, dst, sem):
                    self.args = (src, dst, sem)

                def start(self):
                    pltpu.async_copy(*self.args, add=True)

                def wait(self):
                    pltpu.make_async_copy(*self.args).wait()

            def sc_data(ci, b):
                return _Cp(data_v.at[b], acc_sh.at[idx_v.at[ci]],
                           sc_sems.at[b])

            def sc_cnt(ci):
                return _Cp(ones_v, cnt_sh.at[idx_v.at[ci]], cnt_sem)

            ld(0, 0).start()
            ld(1, 1).start()

            def mk_body(nfull):
                def body(ci, carry):
                    b = ci & 3
                    ld(ci, b).wait()
                    sc_data(ci, b).start()
                    sc_cnt(ci).start()
                    nxt = ci + 2
                    b2 = nxt & 3

                    @pl.when(jnp.logical_and(nxt < nfull, ci >= 2))
                    def _():
                        sc_data(ci - 2, b2).wait()
                        sc_cnt(ci - 2).wait()
                        ld(nxt, b2).start()

                    @pl.when(jnp.logical_and(nxt < nfull, ci < 2))
                    def _():
                        ld(nxt, b2).start()

                    return carry
                return body

            def drain(nfull):
                def dbody(j, carry):
                    sc_data(j, j & 3).wait()
                    sc_cnt(j).wait()
                    return carry
                lax.fori_loop(nfull - 4, nfull, dbody, 0)

            @pl.when(s < TILES - 1)
            def _():
                lax.fori_loop(0, NCH, mk_body(NCH), 0)
                drain(NCH)

            @pl.when(s == TILES - 1)
            def _():
                lax.fori_loop(0, LAST_FULL, mk_body(LAST_FULL), 0)
                drain(LAST_FULL)
                pltpu.sync_copy(
                    x_hbm.at[pl.ds(base + LAST_FULL * CHUNK, LAST_REM)],
                    data_v.at[0, pl.ds(0, LAST_REM)])
                pltpu.sync_copy(data_v.at[0], acc_sh.at[idx_v.at[LAST_FULL]],
                                add=True)
                pltpu.sync_copy(ones_v, cnt_sh.at[idx_v.at[LAST_FULL]],
                                add=True)

            plsc.subcore_barrier()
            pltpu.sync_copy(acc_sh.at[pl.ds(s * ZROWS, ZROWS)],
                            sum_hbm.at[pl.ds(s * ZROWS, ZROWS)])
            pltpu.sync_copy(cnt_sh.at[pl.ds(s * ZROWS, ZROWS)], c32_v)
            pltpu.sync_copy(c32_v, cnt_hbm.at[pl.ds(s * ZROWS, ZROWS)])

        @pl.when(c == 0)
        def _():
            side(x_s_hbm, ids_s_hbm, sum_s_hbm, cnt_s_hbm)

        @pl.when(c == 1)
        def _():
            side(x_t_hbm, ids_t_hbm, sum_t_hbm, cnt_t_hbm)

    return k(x_s, ids_s, x_t, ids_t, zeros_f, ones_1)


def _tc_mlp(sum_s, cnt_s, sum_t, cnt_t, u, w1t, b1, w2t, b2):
    def body(sum_s_ref, cnt_s_ref, sum_t_ref, cnt_t_ref, u_ref,
             w1_ref, b1_ref, w2_ref, b2_ref, out_ref):
        cs = jnp.maximum(cnt_s_ref[...], 1.0)
        ct = jnp.maximum(cnt_t_ref[...], 1.0)
        ms = sum_s_ref[...] / cs
        mt = sum_t_ref[...] / ct
        h = (jnp.dot(u_ref[...], w1_ref[0:F],
                     preferred_element_type=jnp.float32)
             + jnp.dot(ms, w1_ref[F:2 * F],
                       preferred_element_type=jnp.float32)
             + jnp.dot(mt, w1_ref[2 * F:3 * F],
                       preferred_element_type=jnp.float32)
             + b1_ref[...])
        h = jnp.where(h >= 0, h, 0.1 * h)
        out_ref[...] = (jnp.dot(h, w2_ref[...],
                                preferred_element_type=jnp.float32)
                        + b2_ref[...])

    return pl.pallas_call(
        body,
        out_shape=jax.ShapeDtypeStruct((B, F), jnp.float32),
    )(sum_s, cnt_s, sum_t, cnt_t, u, w1t, b1, w2t, b2)


def kernel(x_s, x_t, edge_index, edge_attr, u, batch_s, batch_t, W1, b1, W2, b2):
    del edge_index, edge_attr
    ids_s = _prep_ids(batch_s)
    ids_t = _prep_ids(batch_t)
    zeros_f = jnp.zeros((ZROWS, F), jnp.float32)
    ones_1 = jnp.ones((CHUNK,), jnp.float32)
    sum_s, cnt_s, sum_t, cnt_t = _sc_segment_sums(
        x_s, ids_s, x_t, ids_t, zeros_f, ones_1)
    return _tc_mlp(sum_s, cnt_s.reshape(B, 1), sum_t, cnt_t.reshape(B, 1), u,
                   W1.T, b1.reshape(1, F), W2.T, b2.reshape(1, F))
